# Initial kernel scaffold; baseline (speedup 1.0000x reference)
import functools

import jax
import jax.numpy as jnp
from jax import lax
from jax.experimental import pallas as pl
from jax.experimental.pallas import tpu as pltpu
from jax.experimental.pallas import tpu_sc as plsc

N = 10000
E = 320000
F_IN = 128
F_OUT = 64

NC = 2
NS = 16
NW = NC * NS
EPW = E // NW
CHUNK = 80
NCHUNK = EPW // CHUNK
RPT = N // NS

_sc_mesh = plsc.VectorSubcoreMesh(
    core_axis_name="c", subcore_axis_name="s", num_cores=NC, num_subcores=NS
)

_Z16 = functools.partial(jnp.zeros, (16,), jnp.float32)


@functools.partial(
    pl.kernel,
    out_type=jax.ShapeDtypeStruct((NC, N), jnp.float32),
    mesh=_sc_mesh,
    scratch_types=[
        pltpu.VMEM((NCHUNK, CHUNK), jnp.int32),
        pltpu.VMEM((NCHUNK, CHUNK), jnp.float32),
        pltpu.VMEM((1008,), jnp.float32),
        pltpu.VMEM_SHARED((N,), jnp.float32),
    ],
)
def _deg_kernel(dst_hbm, ew_hbm, degp_hbm, dst_v, ew_v, zbuf, acc_sh):
    c = lax.axis_index("c")
    s = lax.axis_index("s")
    wid = c * NS + s

    def zb(i, carry):
        zbuf[pl.ds(pl.multiple_of(i * 16, 16), 16)] = _Z16()
        return carry

    lax.fori_loop(0, 63, zb, 0)

    @pl.when(s < 10)
    def _():
        pltpu.sync_copy(
            zbuf.at[pl.ds(0, 1000)],
            acc_sh.at[pl.ds(pl.multiple_of(s * 1000, 8), 1000)],
        )

    plsc.subcore_barrier()

    pltpu.sync_copy(dst_hbm.at[pl.ds(wid * NCHUNK, NCHUNK)], dst_v)
    pltpu.sync_copy(ew_hbm.at[pl.ds(wid * NCHUNK, NCHUNK)], ew_v)

    def body(j, carry):
        pltpu.sync_copy(ew_v.at[j], acc_sh.at[dst_v.at[j]], add=True)
        return carry

    lax.fori_loop(0, NCHUNK, body, 0)
    plsc.subcore_barrier()

    @pl.when(s < 10)
    def _():
        off = pl.multiple_of(s * 1000, 8)
        pltpu.sync_copy(acc_sh.at[pl.ds(off, 1000)], degp_hbm.at[c].at[pl.ds(off, 1000)])


@functools.partial(
    pl.kernel,
    out_type=jax.ShapeDtypeStruct((NC, N, F_OUT), jnp.float32),
    mesh=_sc_mesh,
    scratch_types=[
        pltpu.VMEM((NCHUNK, CHUNK), jnp.int32),
        pltpu.VMEM((NCHUNK, CHUNK), jnp.int32),
        pltpu.VMEM((NCHUNK, CHUNK), jnp.float32),
        pltpu.VMEM((CHUNK, F_OUT), jnp.float32),
        pltpu.VMEM((125, F_OUT), jnp.float32),
        pltpu.VMEM_SHARED((N, F_OUT), jnp.float32),
        pltpu.SemaphoreType.DMA,
    ],
)
def _agg_kernel(src_hbm, dst_hbm, ew_hbm, g_hbm, accp_hbm,
                src_v, dst_v, ew_v, rows_v, zblk, acc_sh, sem):
    c = lax.axis_index("c")
    s = lax.axis_index("s")
    wid = c * NS + s

    def zb(i, carry):
        for q in range(4):
            zblk[i, pl.ds(q * 16, 16)] = _Z16()
        return carry

    lax.fori_loop(0, 125, zb, 0)

    def zc(k, carry):
        pltpu.sync_copy(zblk, acc_sh.at[pl.ds(s * RPT + k * 125, 125)])
        return carry

    lax.fori_loop(0, 5, zc, 0)
    plsc.subcore_barrier()

    pltpu.sync_copy(src_hbm.at[pl.ds(wid * NCHUNK, NCHUNK)], src_v)
    pltpu.sync_copy(dst_hbm.at[pl.ds(wid * NCHUNK, NCHUNK)], dst_v)
    pltpu.sync_copy(ew_hbm.at[pl.ds(wid * NCHUNK, NCHUNK)], ew_v)

    def chunk_body(j, carry):
        pltpu.async_copy(g_hbm.at[src_v.at[j]], rows_v, sem).wait()

        def scale(e, inner):
            w16 = jnp.full((16,), ew_v[j, e], jnp.float32)
            for q in range(4):
                sl = pl.ds(q * 16, 16)
                rows_v[e, sl] = rows_v[e, sl] * w16
            return inner

        lax.fori_loop(0, CHUNK, scale, 0)
        pltpu.sync_copy(rows_v, acc_sh.at[dst_v.at[j]], add=True)
        return carry

    lax.fori_loop(0, NCHUNK, chunk_body, 0)
    plsc.subcore_barrier()

    def fl(k, carry):
        off = s * RPT + k * 125
        pltpu.sync_copy(acc_sh.at[pl.ds(off, 125)], accp_hbm.at[c].at[pl.ds(off, 125)])
        return carry

    lax.fori_loop(0, 5, fl, 0)


def _g_body(x_ref, w_ref, dp_ref, g_ref):
    deg = dp_ref[:, 0:1] + dp_ref[:, 1:2] + 1.0
    dinv = lax.rsqrt(deg)
    h = jnp.dot(x_ref[...], w_ref[...], preferred_element_type=jnp.float32)
    g_ref[...] = h * dinv


def _out_body(a0_ref, a1_ref, g_ref, dp_ref, b_ref, o_ref):
    deg = dp_ref[:, 0:1] + dp_ref[:, 1:2] + 1.0
    dinv = lax.rsqrt(deg)
    o_ref[...] = dinv * (a0_ref[...] + a1_ref[...] + g_ref[...]) + b_ref[...]


_g_call = pl.pallas_call(
    _g_body, out_shape=jax.ShapeDtypeStruct((N, F_OUT), jnp.float32)
)

_out_call = pl.pallas_call(
    _out_body, out_shape=jax.ShapeDtypeStruct((N, F_OUT), jnp.float32)
)


def kernel(x, edge_index, edges_weight, W, b):
    src = edge_index[0].reshape(NW * NCHUNK, CHUNK)
    dst = edge_index[1].reshape(NW * NCHUNK, CHUNK)
    ewr = edges_weight.reshape(NW * NCHUNK, CHUNK)

    degp = _deg_kernel(dst, ewr)
    dpT = degp.T
    g = _g_call(x, W, dpT)
    accp = _agg_kernel(src, dst, ewr, g)
    return _out_call(accp[0], accp[1], g, dpT, b.reshape(1, F_OUT))


# speedup vs baseline: 19.2402x; 19.2402x over previous
import functools

import jax
import jax.numpy as jnp
from jax import lax
from jax.experimental import pallas as pl
from jax.experimental.pallas import tpu as pltpu
from jax.experimental.pallas import tpu_sc as plsc

N = 10000
E = 320000
F_IN = 128
F_OUT = 64

NC = 2
NS = 16
NW = NC * NS
EPW = E // NW
CHUNK = 80
NCHUNK = EPW // CHUNK
RPT = N // NS

_sc_mesh = plsc.VectorSubcoreMesh(
    core_axis_name="c", subcore_axis_name="s", num_cores=NC, num_subcores=NS
)

_Z16 = functools.partial(jnp.zeros, (16,), jnp.float32)


@functools.partial(
    pl.kernel,
    out_type=jax.ShapeDtypeStruct((NC, N), jnp.float32),
    mesh=_sc_mesh,
    scratch_types=[
        pltpu.VMEM((NCHUNK, CHUNK), jnp.int32),
        pltpu.VMEM((NCHUNK, CHUNK), jnp.float32),
        pltpu.VMEM((1008,), jnp.float32),
        pltpu.VMEM_SHARED((N,), jnp.float32),
    ],
    compiler_params=pltpu.CompilerParams(use_tc_tiling_on_sc=False),
)
def _deg_kernel(dst_hbm, ew_hbm, degp_hbm, dst_v, ew_v, zbuf, acc_sh):
    c = lax.axis_index("c")
    s = lax.axis_index("s")
    wid = c * NS + s

    def zb(i, carry):
        zbuf[pl.ds(pl.multiple_of(i * 16, 16), 16)] = _Z16()
        return carry

    lax.fori_loop(0, 63, zb, 0)

    @pl.when(s < 10)
    def _():
        pltpu.sync_copy(
            zbuf.at[pl.ds(0, 1000)],
            acc_sh.at[pl.ds(pl.multiple_of(s * 1000, 8), 1000)],
        )

    plsc.subcore_barrier()

    pltpu.sync_copy(dst_hbm.at[wid], dst_v)
    pltpu.sync_copy(ew_hbm.at[wid], ew_v)

    def body(j, carry):
        pltpu.sync_copy(ew_v.at[j], acc_sh.at[dst_v.at[j]], add=True)
        return carry

    lax.fori_loop(0, NCHUNK, body, 0)
    plsc.subcore_barrier()

    @pl.when(s < 10)
    def _():
        off = pl.multiple_of(s * 1000, 8)
        pltpu.sync_copy(acc_sh.at[pl.ds(off, 1000)], degp_hbm.at[c].at[pl.ds(off, 1000)])


@functools.partial(
    pl.kernel,
    out_type=jax.ShapeDtypeStruct((NC, N, F_OUT), jnp.float32),
    mesh=_sc_mesh,
    scratch_types=[
        pltpu.VMEM((NCHUNK, CHUNK), jnp.int32),
        pltpu.VMEM((NCHUNK, CHUNK), jnp.int32),
        pltpu.VMEM((NCHUNK, CHUNK), jnp.float32),
        pltpu.VMEM((CHUNK, F_OUT), jnp.float32),
        pltpu.VMEM((200, F_OUT), jnp.float32),
        pltpu.VMEM_SHARED((N, F_OUT), jnp.float32),
        pltpu.SemaphoreType.DMA,
    ],
    compiler_params=pltpu.CompilerParams(use_tc_tiling_on_sc=False),
)
def _agg_kernel(src_hbm, dst_hbm, ew_hbm, g_hbm, accp_hbm,
                src_v, dst_v, ew_v, rows_v, zblk, acc_sh, sem):
    c = lax.axis_index("c")
    s = lax.axis_index("s")
    wid = c * NS + s

    def zb(i, carry):
        for q in range(4):
            zblk[i, pl.ds(q * 16, 16)] = _Z16()
        return carry

    lax.fori_loop(0, 200, zb, 0)

    @pl.when(s < 10)
    def _():
        def zc(k, carry):
            off = pl.multiple_of(s * 1000 + k * 200, 8)
            pltpu.sync_copy(zblk, acc_sh.at[pl.ds(off, 200)])
            return carry

        lax.fori_loop(0, 5, zc, 0)

    plsc.subcore_barrier()

    pltpu.sync_copy(src_hbm.at[wid], src_v)
    pltpu.sync_copy(dst_hbm.at[wid], dst_v)
    pltpu.sync_copy(ew_hbm.at[wid], ew_v)

    def chunk_body(j, carry):
        pltpu.async_copy(g_hbm.at[src_v.at[j]], rows_v, sem).wait()

        def scale(eb, inner):
            base = pl.multiple_of(eb * 16, 16)
            ew16 = ew_v[j, pl.ds(base, 16)]
            for t in range(16):
                w16 = jnp.full((16,), ew16[t], jnp.float32)
                for q in range(4):
                    sl = pl.ds(q * 16, 16)
                    rows_v[base + t, sl] = rows_v[base + t, sl] * w16
            return inner

        lax.fori_loop(0, CHUNK // 16, scale, 0)
        pltpu.sync_copy(rows_v, acc_sh.at[dst_v.at[j]], add=True)
        return carry

    lax.fori_loop(0, NCHUNK, chunk_body, 0)
    plsc.subcore_barrier()

    @pl.when(s < 10)
    def _():
        def fl(k, carry):
            off = pl.multiple_of(s * 1000 + k * 200, 8)
            pltpu.sync_copy(acc_sh.at[pl.ds(off, 200)], accp_hbm.at[c].at[pl.ds(off, 200)])
            return carry

        lax.fori_loop(0, 5, fl, 0)


def _g_body(x_ref, w_ref, dp_ref, g_ref):
    deg = dp_ref[:, 0:1] + dp_ref[:, 1:2] + 1.0
    dinv = lax.rsqrt(deg)
    h = jnp.dot(x_ref[...], w_ref[...], preferred_element_type=jnp.float32)
    g_ref[...] = h * dinv


def _out_body(a0_ref, a1_ref, g_ref, dp_ref, b_ref, o_ref):
    deg = dp_ref[:, 0:1] + dp_ref[:, 1:2] + 1.0
    dinv = lax.rsqrt(deg)
    o_ref[...] = dinv * (a0_ref[...] + a1_ref[...] + g_ref[...]) + b_ref[...]


_g_call = pl.pallas_call(
    _g_body, out_shape=jax.ShapeDtypeStruct((N, F_OUT), jnp.float32)
)

_out_call = pl.pallas_call(
    _out_body, out_shape=jax.ShapeDtypeStruct((N, F_OUT), jnp.float32)
)


def kernel(x, edge_index, edges_weight, W, b):
    src = edge_index[0].reshape(NW, NCHUNK, CHUNK)
    dst = edge_index[1].reshape(NW, NCHUNK, CHUNK)
    ewr = edges_weight.reshape(NW, NCHUNK, CHUNK)

    degp = _deg_kernel(dst, ewr)
    dpT = degp.T
    g = _g_call(x, W, dpT)
    accp = _agg_kernel(src, dst, ewr, g)
    return _out_call(accp[0], accp[1], g, dpT, b.reshape(1, F_OUT))


# speedup vs baseline: 41.0094x; 2.1314x over previous
import functools

import jax
import jax.numpy as jnp
from jax import lax
from jax.experimental import pallas as pl
from jax.experimental.pallas import tpu as pltpu
from jax.experimental.pallas import tpu_sc as plsc

N = 10000
E = 320000
F_IN = 128
F_OUT = 64

NC = 2
NS = 16
NW = NC * NS
EPW = E // NW
CHUNK = 80
NCHUNK = EPW // CHUNK
RPT = N // NS
NBUF = 5

_sc_mesh = plsc.VectorSubcoreMesh(
    core_axis_name="c", subcore_axis_name="s", num_cores=NC, num_subcores=NS
)

_Z16 = functools.partial(jnp.zeros, (16,), jnp.float32)


@functools.partial(
    pl.kernel,
    out_type=jax.ShapeDtypeStruct((NC, N), jnp.float32),
    mesh=_sc_mesh,
    scratch_types=[
        pltpu.VMEM((NCHUNK, CHUNK), jnp.int32),
        pltpu.VMEM((NCHUNK, CHUNK), jnp.float32),
        pltpu.VMEM((1008,), jnp.float32),
        pltpu.VMEM_SHARED((N,), jnp.float32),
    ],
    compiler_params=pltpu.CompilerParams(use_tc_tiling_on_sc=False),
)
def _deg_kernel(dst_hbm, ew_hbm, degp_hbm, dst_v, ew_v, zbuf, acc_sh):
    c = lax.axis_index("c")
    s = lax.axis_index("s")
    wid = c * NS + s

    def zb(i, carry):
        zbuf[pl.ds(pl.multiple_of(i * 16, 16), 16)] = _Z16()
        return carry

    lax.fori_loop(0, 63, zb, 0)

    @pl.when(s < 10)
    def _():
        pltpu.sync_copy(
            zbuf.at[pl.ds(0, 1000)],
            acc_sh.at[pl.ds(pl.multiple_of(s * 1000, 8), 1000)],
        )

    plsc.subcore_barrier()

    pltpu.sync_copy(dst_hbm.at[wid], dst_v)
    pltpu.sync_copy(ew_hbm.at[wid], ew_v)

    def body(j, carry):
        pltpu.sync_copy(ew_v.at[j], acc_sh.at[dst_v.at[j]], add=True)
        return carry

    lax.fori_loop(0, NCHUNK, body, 0)
    plsc.subcore_barrier()

    @pl.when(s < 10)
    def _():
        off = pl.multiple_of(s * 1000, 8)
        pltpu.sync_copy(acc_sh.at[pl.ds(off, 1000)], degp_hbm.at[c].at[pl.ds(off, 1000)])


@functools.partial(
    pl.kernel,
    out_type=jax.ShapeDtypeStruct((NC, N, F_OUT), jnp.float32),
    mesh=_sc_mesh,
    scratch_types=[
        pltpu.VMEM((NCHUNK, CHUNK), jnp.int32),
        pltpu.VMEM((NCHUNK, CHUNK), jnp.int32),
        pltpu.VMEM((NCHUNK, CHUNK), jnp.float32),
        pltpu.VMEM((NBUF, CHUNK, F_OUT), jnp.float32),
        pltpu.VMEM((NBUF, CHUNK, F_OUT), jnp.float32),
        pltpu.VMEM_SHARED((N, F_OUT), jnp.float32),
        pltpu.SemaphoreType.DMA,
        pltpu.SemaphoreType.DMA,
    ],
    compiler_params=pltpu.CompilerParams(use_tc_tiling_on_sc=False),
)
def _agg_kernel(src_hbm, dst_hbm, ew_hbm, g_hbm, accp_hbm,
                src_v, dst_v, ew_v, rows_v, msg_v, acc_sh, gsem, ssem):
    c = lax.axis_index("c")
    s = lax.axis_index("s")
    wid = c * NS + s

    def zb(i, carry):
        for q in range(4):
            rows_v[0, i, pl.ds(q * 16, 16)] = _Z16()
        return carry

    lax.fori_loop(0, 40, zb, 0)

    @pl.when(s < 10)
    def _():
        def zc(k, carry):
            off = pl.multiple_of(s * 1000 + k * 40, 8)
            pltpu.sync_copy(rows_v.at[0].at[pl.ds(0, 40)], acc_sh.at[pl.ds(off, 40)])
            return carry

        lax.fori_loop(0, 25, zc, 0)

    plsc.subcore_barrier()

    pltpu.sync_copy(src_hbm.at[wid], src_v)
    pltpu.sync_copy(dst_hbm.at[wid], dst_v)
    pltpu.sync_copy(ew_hbm.at[wid], ew_v)

    def group_body(g, carry):
        for b in range(NBUF):
            j = g * NBUF + b
            pltpu.async_copy(g_hbm.at[src_v.at[j]], rows_v.at[b], gsem)
        for b in range(NBUF):
            j = g * NBUF + b
            pltpu.make_async_copy(g_hbm.at[src_v.at[j]], rows_v.at[b], gsem).wait()

        @pl.when(g > 0)
        def _():
            for b in range(NBUF):
                jp = (g - 1) * NBUF + b
                pltpu.make_async_copy(
                    msg_v.at[b], acc_sh.at[dst_v.at[jp]], ssem
                ).wait()

        for b in range(NBUF):
            j = g * NBUF + b

            def scale(eb, inner, b=b, j=j):
                base = pl.multiple_of(eb * 16, 16)
                ew16 = ew_v[j, pl.ds(base, 16)]
                for t in range(16):
                    w16 = jnp.full((16,), ew16[t], jnp.float32)
                    for q in range(4):
                        sl = pl.ds(q * 16, 16)
                        msg_v[b, base + t, sl] = rows_v[b, base + t, sl] * w16
                return inner

            lax.fori_loop(0, CHUNK // 16, scale, 0)
        for b in range(NBUF):
            j = g * NBUF + b
            pltpu.async_copy(msg_v.at[b], acc_sh.at[dst_v.at[j]], ssem, add=True)
        return carry

    lax.fori_loop(0, NCHUNK // NBUF, group_body, 0)
    for b in range(NBUF):
        jl = NCHUNK - NBUF + b
        pltpu.make_async_copy(msg_v.at[b], acc_sh.at[dst_v.at[jl]], ssem).wait()
    plsc.subcore_barrier()

    @pl.when(s < 10)
    def _():
        def fl(k, carry):
            off = pl.multiple_of(s * 1000 + k * 200, 8)
            pltpu.sync_copy(acc_sh.at[pl.ds(off, 200)], accp_hbm.at[c].at[pl.ds(off, 200)])
            return carry

        lax.fori_loop(0, 5, fl, 0)


def _g_body(x_ref, w_ref, dp_ref, g_ref):
    deg = dp_ref[:, 0:1] + dp_ref[:, 1:2] + 1.0
    dinv = lax.rsqrt(deg)
    h = jnp.dot(x_ref[...], w_ref[...], preferred_element_type=jnp.float32)
    g_ref[...] = h * dinv


def _out_body(a0_ref, a1_ref, g_ref, dp_ref, b_ref, o_ref):
    deg = dp_ref[:, 0:1] + dp_ref[:, 1:2] + 1.0
    dinv = lax.rsqrt(deg)
    o_ref[...] = dinv * (a0_ref[...] + a1_ref[...] + g_ref[...]) + b_ref[...]


_g_call = pl.pallas_call(
    _g_body, out_shape=jax.ShapeDtypeStruct((N, F_OUT), jnp.float32)
)

_out_call = pl.pallas_call(
    _out_body, out_shape=jax.ShapeDtypeStruct((N, F_OUT), jnp.float32)
)


def kernel(x, edge_index, edges_weight, W, b):
    src = edge_index[0].reshape(NW, NCHUNK, CHUNK)
    dst = edge_index[1].reshape(NW, NCHUNK, CHUNK)
    ewr = edges_weight.reshape(NW, NCHUNK, CHUNK)

    degp = _deg_kernel(dst, ewr)
    dpT = degp.T
    g = _g_call(x, W, dpT)
    accp = _agg_kernel(src, dst, ewr, g)
    return _out_call(accp[0], accp[1], g, dpT, b.reshape(1, F_OUT))


# speedup vs baseline: 42.5905x; 1.0386x over previous
import functools

import jax
import jax.numpy as jnp
from jax import lax
from jax.experimental import pallas as pl
from jax.experimental.pallas import tpu as pltpu
from jax.experimental.pallas import tpu_sc as plsc

N = 10000
E = 320000
F_IN = 128
F_OUT = 64

NC = 2
NS = 16
NW = NC * NS
EPW = E // NW
CHUNK = 80
NCHUNK = EPW // CHUNK
RPT = N // NS
NBUF = 5

_sc_mesh = plsc.VectorSubcoreMesh(
    core_axis_name="c", subcore_axis_name="s", num_cores=NC, num_subcores=NS
)

_Z16 = functools.partial(jnp.zeros, (16,), jnp.float32)


@functools.partial(
    pl.kernel,
    out_type=jax.ShapeDtypeStruct((NC, N), jnp.float32),
    mesh=_sc_mesh,
    scratch_types=[
        pltpu.VMEM((NCHUNK, CHUNK), jnp.int32),
        pltpu.VMEM((NCHUNK, CHUNK), jnp.float32),
        pltpu.VMEM((1008,), jnp.float32),
        pltpu.VMEM_SHARED((N,), jnp.float32),
    ],
    compiler_params=pltpu.CompilerParams(use_tc_tiling_on_sc=False),
)
def _deg_kernel(dst_hbm, ew_hbm, degp_hbm, dst_v, ew_v, zbuf, acc_sh):
    c = lax.axis_index("c")
    s = lax.axis_index("s")
    wid = c * NS + s

    def zb(i, carry):
        zbuf[pl.ds(pl.multiple_of(i * 16, 16), 16)] = _Z16()
        return carry

    lax.fori_loop(0, 63, zb, 0)

    @pl.when(s < 10)
    def _():
        pltpu.sync_copy(
            zbuf.at[pl.ds(0, 1000)],
            acc_sh.at[pl.ds(pl.multiple_of(s * 1000, 8), 1000)],
        )

    plsc.subcore_barrier()

    pltpu.sync_copy(dst_hbm.at[wid], dst_v)
    pltpu.sync_copy(ew_hbm.at[wid], ew_v)

    def body(j, carry):
        pltpu.sync_copy(ew_v.at[j], acc_sh.at[dst_v.at[j]], add=True)
        return carry

    lax.fori_loop(0, NCHUNK, body, 0)
    plsc.subcore_barrier()

    @pl.when(s < 10)
    def _():
        off = pl.multiple_of(s * 1000, 8)
        pltpu.sync_copy(acc_sh.at[pl.ds(off, 1000)], degp_hbm.at[c].at[pl.ds(off, 1000)])


@functools.partial(
    pl.kernel,
    out_type=(
        jax.ShapeDtypeStruct((N, F_OUT), jnp.float32),
        jax.ShapeDtypeStruct((N, F_OUT), jnp.float32),
    ),
    mesh=_sc_mesh,
    scratch_types=[
        pltpu.VMEM((NCHUNK, CHUNK), jnp.int32),
        pltpu.VMEM((NCHUNK, CHUNK), jnp.int32),
        pltpu.VMEM((NCHUNK, CHUNK), jnp.float32),
        pltpu.VMEM((NBUF, CHUNK, F_OUT), jnp.float32),
        pltpu.VMEM((NBUF, CHUNK, F_OUT), jnp.float32),
        pltpu.VMEM_SHARED((N, F_OUT), jnp.float32),
        pltpu.SemaphoreType.DMA,
        pltpu.SemaphoreType.DMA,
    ],
    compiler_params=pltpu.CompilerParams(use_tc_tiling_on_sc=False),
)
def _agg_kernel(src_hbm, dst_hbm, ew_hbm, g_hbm, a0_hbm, a1_hbm,
                src_v, dst_v, ew_v, rows_v, msg_v, acc_sh, gsem, ssem):
    c = lax.axis_index("c")
    s = lax.axis_index("s")
    wid = c * NS + s

    def zb(i, carry):
        for q in range(4):
            rows_v[0, i, pl.ds(q * 16, 16)] = _Z16()
        return carry

    lax.fori_loop(0, 40, zb, 0)

    @pl.when(s < 10)
    def _():
        def zc(k, carry):
            off = pl.multiple_of(s * 1000 + k * 40, 8)
            pltpu.sync_copy(rows_v.at[0].at[pl.ds(0, 40)], acc_sh.at[pl.ds(off, 40)])
            return carry

        lax.fori_loop(0, 25, zc, 0)

    plsc.subcore_barrier()

    pltpu.sync_copy(src_hbm.at[wid], src_v)
    pltpu.sync_copy(dst_hbm.at[wid], dst_v)
    pltpu.sync_copy(ew_hbm.at[wid], ew_v)

    def group_body(g, carry):
        for b in range(NBUF):
            j = g * NBUF + b
            pltpu.async_copy(g_hbm.at[src_v.at[j]], rows_v.at[b], gsem)
        for b in range(NBUF):
            j = g * NBUF + b
            pltpu.make_async_copy(g_hbm.at[src_v.at[j]], rows_v.at[b], gsem).wait()

        @pl.when(g > 0)
        def _():
            for b in range(NBUF):
                jp = (g - 1) * NBUF + b
                pltpu.make_async_copy(
                    msg_v.at[b], acc_sh.at[dst_v.at[jp]], ssem
                ).wait()

        for b in range(NBUF):
            j = g * NBUF + b

            def scale(eb, inner, b=b, j=j):
                base = pl.multiple_of(eb * 16, 16)
                ew16 = ew_v[j, pl.ds(base, 16)]
                for t in range(16):
                    w16 = jnp.full((16,), ew16[t], jnp.float32)
                    for q in range(4):
                        sl = pl.ds(q * 16, 16)
                        msg_v[b, base + t, sl] = rows_v[b, base + t, sl] * w16
                return inner

            lax.fori_loop(0, CHUNK // 16, scale, 0)
        for b in range(NBUF):
            j = g * NBUF + b
            pltpu.async_copy(msg_v.at[b], acc_sh.at[dst_v.at[j]], ssem, add=True)
        return carry

    lax.fori_loop(0, NCHUNK // NBUF, group_body, 0)
    for b in range(NBUF):
        jl = NCHUNK - NBUF + b
        pltpu.make_async_copy(msg_v.at[b], acc_sh.at[dst_v.at[jl]], ssem).wait()
    plsc.subcore_barrier()

    @pl.when(jnp.logical_and(s < 10, c == 0))
    def _():
        def fl(k, carry):
            off = pl.multiple_of(s * 1000 + k * 200, 8)
            pltpu.sync_copy(acc_sh.at[pl.ds(off, 200)], a0_hbm.at[pl.ds(off, 200)])
            return carry

        lax.fori_loop(0, 5, fl, 0)

    @pl.when(jnp.logical_and(s < 10, c == 1))
    def _():
        def fl(k, carry):
            off = pl.multiple_of(s * 1000 + k * 200, 8)
            pltpu.sync_copy(acc_sh.at[pl.ds(off, 200)], a1_hbm.at[pl.ds(off, 200)])
            return carry

        lax.fori_loop(0, 5, fl, 0)


def _h_body(x_ref, w_ref, h_ref):
    h_ref[...] = jnp.dot(x_ref[...], w_ref[...], preferred_element_type=jnp.float32)


def _g_body(h_ref, dp_ref, g_ref):
    deg = dp_ref[:, 0:1] + dp_ref[:, 1:2] + 1.0
    dinv = lax.rsqrt(deg)
    g_ref[...] = h_ref[...] * dinv


def _out_body(a0_ref, a1_ref, g_ref, dp_ref, b_ref, o_ref):
    deg = dp_ref[:, 0:1] + dp_ref[:, 1:2] + 1.0
    dinv = lax.rsqrt(deg)
    o_ref[...] = dinv * (a0_ref[...] + a1_ref[...] + g_ref[...]) + b_ref[...]


_h_call = pl.pallas_call(
    _h_body, out_shape=jax.ShapeDtypeStruct((N, F_OUT), jnp.float32)
)

_g_call = pl.pallas_call(
    _g_body, out_shape=jax.ShapeDtypeStruct((N, F_OUT), jnp.float32)
)

_out_call = pl.pallas_call(
    _out_body, out_shape=jax.ShapeDtypeStruct((N, F_OUT), jnp.float32)
)


def kernel(x, edge_index, edges_weight, W, b):
    src = edge_index[0].reshape(NW, NCHUNK, CHUNK)
    dst = edge_index[1].reshape(NW, NCHUNK, CHUNK)
    ewr = edges_weight.reshape(NW, NCHUNK, CHUNK)

    h = _h_call(x, W)
    degp = _deg_kernel(dst, ewr)
    dpT = degp.T
    g = _g_call(h, dpT)
    a0, a1 = _agg_kernel(src, dst, ewr, g)
    return _out_call(a0, a1, g, dpT, b.reshape(1, F_OUT))
